# baseline (speedup 1.0000x reference)
import jax
import jax.numpy as jnp
from jax.experimental import pallas as pl
from jax.experimental.pallas import tpu as pltpu

_HOP = 256
_K_CHUNK = 1024


def _dot(a, b):
    return jax.lax.dot_general(
        a.astype(jnp.bfloat16), b.astype(jnp.bfloat16),
        (((1,), (0,)), ((), ())),
        preferred_element_type=jnp.float32)


def _vq_kernel(xt_ref, w1_ref, w2_ref, w3_ref, cbt_ref, out_ref,
               zb_ref, zn_ref, bv_ref, bi_ref):
    k = pl.program_id(0)
    n_k = pl.num_programs(0)

    @pl.when(k == 0)
    def _encoder():
        _, B, L2 = xt_ref.shape

        def _frame_sum(j):
            base = j * _HOP

            def body(t, acc):
                lo = jnp.abs(xt_ref[pl.ds(base + 8 * t, 8)])
                hi = jnp.abs(xt_ref[pl.ds(base + 128 + 8 * t, 8)])
                return acc + (lo + hi)

            acc = jax.lax.fori_loop(0, 16, body,
                                    jnp.zeros((8, B, L2), jnp.float32))
            b1 = acc[0:4] + acc[4:8]
            b2 = b1[0:2] + b1[2:4]
            fm = b2[0] + b2[1]
            return jnp.log1p(fm * (1.0 / _HOP))

        a, b, c, d = (_frame_sum(j) for j in range(4))

        def _sum512(parts):
            comps = []
            for p in parts:
                v = ((p[:, 0:32] + p[:, 32:64]) + p[:, 64:96]) + p[:, 96:128]
                acc = v[:, 0:2]
                for t in range(1, 16):
                    acc = acc + v[:, 2 * t:2 * t + 2]
                comps.append(acc[:, 0:1] + acc[:, 1:2])
            return (comps[0] + comps[2]) + (comps[1] + comps[3])

        mu = _sum512([a, b, c, d]) * (1.0 / 512)
        da, db, dc, dd = a - mu, b - mu, c - mu, d - mu
        var = _sum512([da * da, db * db, dc * dc, dd * dd]) * (1.0 / 512)
        sd = jnp.sqrt(var) + 1e-5
        a, b, c, d = da / sd, db / sd, dc / sd, dd / sd
        a_nxt = jnp.concatenate([a[:, 1:], jnp.zeros((B, 1), jnp.float32)],
                                axis=1)

        bf = lambda v: v.astype(jnp.bfloat16).astype(jnp.float32)
        w1 = bf(w1_ref[...])
        a, b, c, d, a_nxt = bf(a), bf(b), bf(c), bf(d), bf(a_nxt)
        h1e = jax.nn.relu(a[..., None] * w1[0] + b[..., None] * w1[1]
                          + c[..., None] * w1[2])
        h1o = jax.nn.relu(c[..., None] * w1[0] + d[..., None] * w1[1]
                          + a_nxt[..., None] * w1[2])

        H = h1e.shape[-1]
        h1e_nxt = jnp.concatenate(
            [h1e[:, 1:, :], jnp.zeros((B, 1, H), jnp.float32)], axis=1)
        xcat = jnp.concatenate([h1e.reshape(B * L2, H), h1o.reshape(B * L2, H),
                                h1e_nxt.reshape(B * L2, H)], axis=1)
        h2 = jax.nn.relu(_dot(xcat, w2_ref[...]))

        z = _dot(h2, w3_ref[...])
        zsq = z * z
        acc = zsq[:, 0:8]
        for t in range(1, 8):
            acc = acc + zsq[:, 8 * t:8 * t + 8]
        acc = acc[:, 0:4] + acc[:, 4:8]
        acc = acc[:, 0:2] + acc[:, 2:4]
        zn_ref[...] = acc[:, 0:1] + acc[:, 1:2]
        zb_ref[...] = z.astype(jnp.bfloat16)
        bv_ref[...] = jnp.full(bv_ref.shape, jnp.inf, jnp.float32)
        bi_ref[...] = jnp.zeros(bi_ref.shape, jnp.int32)

    cb_c = cbt_ref[...]
    cbn = jnp.sum(cb_c * cb_c, axis=0)
    t = jax.lax.dot_general(zb_ref[...], cb_c.astype(jnp.bfloat16),
                            (((1,), (0,)), ((), ())),
                            preferred_element_type=jnp.float32)
    dist = (zn_ref[...] - 2.0 * t) + cbn[None, :]
    v = jnp.min(dist, axis=1, keepdims=True)
    idx = (jnp.argmin(dist, axis=1).astype(jnp.int32)
           + k * _K_CHUNK).reshape(v.shape)
    take_new = v < bv_ref[...]
    bv_ref[...] = jnp.where(take_new, v, bv_ref[...])
    bi_ref[...] = jnp.where(take_new, idx, bi_ref[...])

    @pl.when(k == n_k - 1)
    def _emit():
        out_ref[...] = bi_ref[...].reshape(out_ref.shape)


def kernel(x, W1, W2, W3, codebook):
    B, T = x.shape
    L2 = T // _HOP // 4
    xt = x.reshape(B, L2, 4, _HOP).transpose(2, 3, 0, 1).reshape(4 * _HOP, B, L2)
    w1 = jnp.transpose(W1[:, 0, :], (1, 0))
    w2 = jnp.transpose(W2, (2, 1, 0)).reshape(-1, W2.shape[0])
    w3 = jnp.transpose(W3[:, :, 0], (1, 0))
    cbt = jnp.transpose(codebook, (1, 0))
    H = w1.shape[1]
    D, K = cbt.shape
    n = B * L2
    return pl.pallas_call(
        _vq_kernel,
        grid=(K // _K_CHUNK,),
        in_specs=[
            pl.BlockSpec((4 * _HOP, B, L2), lambda k: (0, 0, 0)),
            pl.BlockSpec((3, H), lambda k: (0, 0)),
            pl.BlockSpec((3 * H, H), lambda k: (0, 0)),
            pl.BlockSpec((H, D), lambda k: (0, 0)),
            pl.BlockSpec((D, _K_CHUNK), lambda k: (0, k)),
        ],
        out_specs=pl.BlockSpec((B, L2), lambda k: (0, 0)),
        out_shape=jax.ShapeDtypeStruct((B, L2), jnp.int32),
        scratch_shapes=[
            pltpu.VMEM((n, D), jnp.bfloat16),
            pltpu.VMEM((n, 1), jnp.float32),
            pltpu.VMEM((n, 1), jnp.float32),
            pltpu.VMEM((n, 1), jnp.int32),
        ],
    )(xt, w1, w2, w3, cbt)


# speedup vs baseline: 1.1644x; 1.1644x over previous
import jax
import jax.numpy as jnp
from jax.experimental import pallas as pl
from jax.experimental.pallas import tpu as pltpu

_HOP = 256
_K_CHUNK = 2048


def _dot(a, b):
    return jax.lax.dot_general(
        a.astype(jnp.bfloat16), b.astype(jnp.bfloat16),
        (((1,), (0,)), ((), ())),
        preferred_element_type=jnp.float32)


def _vq_kernel(xt_ref, w1_ref, w2_ref, w3_ref, cbt_ref, out_ref,
               zb_ref, zn_ref, bv_ref, bi_ref):
    k = pl.program_id(0)
    n_k = pl.num_programs(0)

    @pl.when(k == 0)
    def _encoder():
        _, B, L2 = xt_ref.shape

        def _frame_sum(j):
            base = j * _HOP

            def body(t, acc):
                lo = jnp.abs(xt_ref[pl.ds(base + 8 * t, 8)])
                hi = jnp.abs(xt_ref[pl.ds(base + 128 + 8 * t, 8)])
                return acc + (lo + hi)

            acc = jax.lax.fori_loop(0, 16, body,
                                    jnp.zeros((8, B, L2), jnp.float32))
            b1 = acc[0:4] + acc[4:8]
            b2 = b1[0:2] + b1[2:4]
            fm = b2[0] + b2[1]
            return jnp.log1p(fm * (1.0 / _HOP))

        a, b, c, d = (_frame_sum(j) for j in range(4))

        def _sum512(parts):
            comps = []
            for p in parts:
                v = ((p[:, 0:32] + p[:, 32:64]) + p[:, 64:96]) + p[:, 96:128]
                acc = v[:, 0:2]
                for t in range(1, 16):
                    acc = acc + v[:, 2 * t:2 * t + 2]
                comps.append(acc[:, 0:1] + acc[:, 1:2])
            return (comps[0] + comps[2]) + (comps[1] + comps[3])

        mu = _sum512([a, b, c, d]) * (1.0 / 512)
        da, db, dc, dd = a - mu, b - mu, c - mu, d - mu
        var = _sum512([da * da, db * db, dc * dc, dd * dd]) * (1.0 / 512)
        sd = jnp.sqrt(var) + 1e-5
        a, b, c, d = da / sd, db / sd, dc / sd, dd / sd
        a_nxt = jnp.concatenate([a[:, 1:], jnp.zeros((B, 1), jnp.float32)],
                                axis=1)

        bf = lambda v: v.astype(jnp.bfloat16).astype(jnp.float32)
        w1 = bf(w1_ref[...])
        a, b, c, d, a_nxt = bf(a), bf(b), bf(c), bf(d), bf(a_nxt)
        h1e = jax.nn.relu(a[..., None] * w1[0] + b[..., None] * w1[1]
                          + c[..., None] * w1[2])
        h1o = jax.nn.relu(c[..., None] * w1[0] + d[..., None] * w1[1]
                          + a_nxt[..., None] * w1[2])

        H = h1e.shape[-1]
        h1e_nxt = jnp.concatenate(
            [h1e[:, 1:, :], jnp.zeros((B, 1, H), jnp.float32)], axis=1)
        xcat = jnp.concatenate([h1e.reshape(B * L2, H), h1o.reshape(B * L2, H),
                                h1e_nxt.reshape(B * L2, H)], axis=1)
        h2 = jax.nn.relu(_dot(xcat, w2_ref[...]))

        z = _dot(h2, w3_ref[...])
        zsq = z * z
        acc = zsq[:, 0:8]
        for t in range(1, 8):
            acc = acc + zsq[:, 8 * t:8 * t + 8]
        acc = acc[:, 0:4] + acc[:, 4:8]
        acc = acc[:, 0:2] + acc[:, 2:4]
        zn_ref[...] = acc[:, 0:1] + acc[:, 1:2]
        zb_ref[...] = z.astype(jnp.bfloat16)
        bv_ref[...] = jnp.full(bv_ref.shape, jnp.inf, jnp.float32)
        bi_ref[...] = jnp.zeros(bi_ref.shape, jnp.int32)

    cb_c = cbt_ref[...]
    cbn = jnp.sum(cb_c * cb_c, axis=0)
    t = jax.lax.dot_general(zb_ref[...], cb_c.astype(jnp.bfloat16),
                            (((1,), (0,)), ((), ())),
                            preferred_element_type=jnp.float32)
    dist = (zn_ref[...] - 2.0 * t) + cbn[None, :]
    v = jnp.min(dist, axis=1, keepdims=True)
    idx = (jnp.argmin(dist, axis=1).astype(jnp.int32)
           + k * _K_CHUNK).reshape(v.shape)
    take_new = v < bv_ref[...]
    bv_ref[...] = jnp.where(take_new, v, bv_ref[...])
    bi_ref[...] = jnp.where(take_new, idx, bi_ref[...])

    @pl.when(k == n_k - 1)
    def _emit():
        out_ref[...] = bi_ref[...].reshape(out_ref.shape)


def kernel(x, W1, W2, W3, codebook):
    B, T = x.shape
    L2 = T // _HOP // 4
    xt = x.reshape(B, L2, 4, _HOP).transpose(2, 3, 0, 1).reshape(4 * _HOP, B, L2)
    w1 = jnp.transpose(W1[:, 0, :], (1, 0))
    w2 = jnp.transpose(W2, (2, 1, 0)).reshape(-1, W2.shape[0])
    w3 = jnp.transpose(W3[:, :, 0], (1, 0))
    cbt = jnp.transpose(codebook, (1, 0))
    H = w1.shape[1]
    D, K = cbt.shape
    n = B * L2
    return pl.pallas_call(
        _vq_kernel,
        grid=(K // _K_CHUNK,),
        in_specs=[
            pl.BlockSpec((4 * _HOP, B, L2), lambda k: (0, 0, 0)),
            pl.BlockSpec((3, H), lambda k: (0, 0)),
            pl.BlockSpec((3 * H, H), lambda k: (0, 0)),
            pl.BlockSpec((H, D), lambda k: (0, 0)),
            pl.BlockSpec((D, _K_CHUNK), lambda k: (0, k)),
        ],
        out_specs=pl.BlockSpec((B, L2), lambda k: (0, 0)),
        out_shape=jax.ShapeDtypeStruct((B, L2), jnp.int32),
        scratch_shapes=[
            pltpu.VMEM((n, D), jnp.bfloat16),
            pltpu.VMEM((n, 1), jnp.float32),
            pltpu.VMEM((n, 1), jnp.float32),
            pltpu.VMEM((n, 1), jnp.int32),
        ],
    )(xt, w1, w2, w3, cbt)


# speedup vs baseline: 1.2575x; 1.0800x over previous
import jax
import jax.numpy as jnp
from jax.experimental import pallas as pl
from jax.experimental.pallas import tpu as pltpu

_HOP = 256
_K_CHUNK = 4096


def _dot(a, b):
    return jax.lax.dot_general(
        a.astype(jnp.bfloat16), b.astype(jnp.bfloat16),
        (((1,), (0,)), ((), ())),
        preferred_element_type=jnp.float32)


def _vq_kernel(xt_ref, w1_ref, w2_ref, w3_ref, cbt_ref, out_ref,
               zb_ref, zn_ref, bv_ref, bi_ref):
    k = pl.program_id(0)
    n_k = pl.num_programs(0)

    @pl.when(k == 0)
    def _encoder():
        _, B, L2 = xt_ref.shape

        def _frame_sum(j):
            base = j * _HOP

            def body(t, acc):
                lo = jnp.abs(xt_ref[pl.ds(base + 8 * t, 8)])
                hi = jnp.abs(xt_ref[pl.ds(base + 128 + 8 * t, 8)])
                return acc + (lo + hi)

            acc = jax.lax.fori_loop(0, 16, body,
                                    jnp.zeros((8, B, L2), jnp.float32))
            b1 = acc[0:4] + acc[4:8]
            b2 = b1[0:2] + b1[2:4]
            fm = b2[0] + b2[1]
            return jnp.log1p(fm * (1.0 / _HOP))

        a, b, c, d = (_frame_sum(j) for j in range(4))

        def _sum512(parts):
            comps = []
            for p in parts:
                v = ((p[:, 0:32] + p[:, 32:64]) + p[:, 64:96]) + p[:, 96:128]
                acc = v[:, 0:2]
                for t in range(1, 16):
                    acc = acc + v[:, 2 * t:2 * t + 2]
                comps.append(acc[:, 0:1] + acc[:, 1:2])
            return (comps[0] + comps[2]) + (comps[1] + comps[3])

        mu = _sum512([a, b, c, d]) * (1.0 / 512)
        da, db, dc, dd = a - mu, b - mu, c - mu, d - mu
        var = _sum512([da * da, db * db, dc * dc, dd * dd]) * (1.0 / 512)
        sd = jnp.sqrt(var) + 1e-5
        a, b, c, d = da / sd, db / sd, dc / sd, dd / sd
        a_nxt = jnp.concatenate([a[:, 1:], jnp.zeros((B, 1), jnp.float32)],
                                axis=1)

        bf = lambda v: v.astype(jnp.bfloat16).astype(jnp.float32)
        w1 = bf(w1_ref[...])
        a, b, c, d, a_nxt = bf(a), bf(b), bf(c), bf(d), bf(a_nxt)
        h1e = jax.nn.relu(a[..., None] * w1[0] + b[..., None] * w1[1]
                          + c[..., None] * w1[2])
        h1o = jax.nn.relu(c[..., None] * w1[0] + d[..., None] * w1[1]
                          + a_nxt[..., None] * w1[2])

        H = h1e.shape[-1]
        h1e_nxt = jnp.concatenate(
            [h1e[:, 1:, :], jnp.zeros((B, 1, H), jnp.float32)], axis=1)
        xcat = jnp.concatenate([h1e.reshape(B * L2, H), h1o.reshape(B * L2, H),
                                h1e_nxt.reshape(B * L2, H)], axis=1)
        h2 = jax.nn.relu(_dot(xcat, w2_ref[...]))

        z = _dot(h2, w3_ref[...])
        zsq = z * z
        acc = zsq[:, 0:8]
        for t in range(1, 8):
            acc = acc + zsq[:, 8 * t:8 * t + 8]
        acc = acc[:, 0:4] + acc[:, 4:8]
        acc = acc[:, 0:2] + acc[:, 2:4]
        zn_ref[...] = acc[:, 0:1] + acc[:, 1:2]
        zb_ref[...] = z.astype(jnp.bfloat16)
        bv_ref[...] = jnp.full(bv_ref.shape, jnp.inf, jnp.float32)
        bi_ref[...] = jnp.zeros(bi_ref.shape, jnp.int32)

    cb_c = cbt_ref[...]
    cbn = jnp.sum(cb_c * cb_c, axis=0)
    t = jax.lax.dot_general(zb_ref[...], cb_c.astype(jnp.bfloat16),
                            (((1,), (0,)), ((), ())),
                            preferred_element_type=jnp.float32)
    dist = (zn_ref[...] - 2.0 * t) + cbn[None, :]
    v = jnp.min(dist, axis=1, keepdims=True)
    idx = (jnp.argmin(dist, axis=1).astype(jnp.int32)
           + k * _K_CHUNK).reshape(v.shape)
    take_new = v < bv_ref[...]
    bv_ref[...] = jnp.where(take_new, v, bv_ref[...])
    bi_ref[...] = jnp.where(take_new, idx, bi_ref[...])

    @pl.when(k == n_k - 1)
    def _emit():
        out_ref[...] = bi_ref[...].reshape(out_ref.shape)


def kernel(x, W1, W2, W3, codebook):
    B, T = x.shape
    L2 = T // _HOP // 4
    xt = x.reshape(B, L2, 4, _HOP).transpose(2, 3, 0, 1).reshape(4 * _HOP, B, L2)
    w1 = jnp.transpose(W1[:, 0, :], (1, 0))
    w2 = jnp.transpose(W2, (2, 1, 0)).reshape(-1, W2.shape[0])
    w3 = jnp.transpose(W3[:, :, 0], (1, 0))
    cbt = jnp.transpose(codebook, (1, 0))
    H = w1.shape[1]
    D, K = cbt.shape
    n = B * L2
    return pl.pallas_call(
        _vq_kernel,
        grid=(K // _K_CHUNK,),
        in_specs=[
            pl.BlockSpec((4 * _HOP, B, L2), lambda k: (0, 0, 0)),
            pl.BlockSpec((3, H), lambda k: (0, 0)),
            pl.BlockSpec((3 * H, H), lambda k: (0, 0)),
            pl.BlockSpec((H, D), lambda k: (0, 0)),
            pl.BlockSpec((D, _K_CHUNK), lambda k: (0, k)),
        ],
        out_specs=pl.BlockSpec((B, L2), lambda k: (0, 0)),
        out_shape=jax.ShapeDtypeStruct((B, L2), jnp.int32),
        scratch_shapes=[
            pltpu.VMEM((n, D), jnp.bfloat16),
            pltpu.VMEM((n, 1), jnp.float32),
            pltpu.VMEM((n, 1), jnp.float32),
            pltpu.VMEM((n, 1), jnp.int32),
        ],
    )(xt, w1, w2, w3, cbt)
